# baseline (speedup 1.0000x reference)
import dataclasses
import functools

import jax
import jax.numpy as jnp
from jax import lax
from jax.experimental import pallas as pl
from jax.experimental.pallas import tpu as pltpu
from jax.experimental.pallas import tpu_sc as plsc

N_NODES = 10000
D = 128
HIDDEN = 128
N_CLASSES = 40

NC = 2
NS = 16
NW = NC * NS
K = 128
CH = 40
NSPLIT = 4
QS = K // NSPLIT
N_PAD = 10240
ROWS_PT = N_PAD // NS
R_BLK = 2560

_MESH = plsc.VectorSubcoreMesh(core_axis_name="c", subcore_axis_name="s")

_CP_NO_LAYOUT = pltpu.CompilerParams()
if "needs_layout_passes" in pltpu.CompilerParams.__dataclass_fields__:
    _CP_NO_LAYOUT = dataclasses.replace(_CP_NO_LAYOUT, needs_layout_passes=False)


def _sc_degree(dst_w):
    C = dst_w.shape[1]

    @functools.partial(
        pl.kernel,
        out_type=jax.ShapeDtypeStruct((NW, N_PAD), jnp.float32),
        mesh=_MESH,
        compiler_params=_CP_NO_LAYOUT,
        scratch_types=[
            pltpu.VMEM((N_PAD,), jnp.float32),
            pltpu.VMEM((C, K), jnp.int32),
        ],
    )
    def deg_kernel(dst_hbm, out_hbm, deg_v, idx_v):
        cid = lax.axis_index("c")
        sid = lax.axis_index("s")
        wid = sid * NC + cid
        pltpu.sync_copy(dst_hbm.at[wid], idx_v)

        @pl.loop(0, N_PAD // 16)
        def _(i):
            deg_v[pl.ds(i * 16, 16)] = jnp.zeros((16,), jnp.float32)

        ones = jnp.ones((16,), jnp.float32)

        @pl.loop(0, C)
        def _(c):
            for j in range(K // 16):
                idx = idx_v[c, pl.ds(j * 16, 16)]
                plsc.addupdate_scatter(deg_v, [idx], ones)

        pltpu.sync_copy(deg_v, out_hbm.at[wid])

    return deg_kernel(dst_w)


def _sc_aggregate(h_pad, src_w, dst_w, zeros128):
    C = src_w.shape[1]
    assert C == 2 * CH

    @functools.partial(
        pl.kernel,
        out_type=jax.ShapeDtypeStruct((NC, N_PAD, D), jnp.float32),
        mesh=_MESH,
        scratch_types=[
            pltpu.VMEM_SHARED((N_PAD, D), jnp.float32),
            pltpu.VMEM((CH, K), jnp.int32),
            pltpu.VMEM((CH, K), jnp.int32),
            pltpu.VMEM((K, D), jnp.float32),
            pltpu.VMEM((K, D), jnp.float32),
            pltpu.SemaphoreType.DMA,
            pltpu.SemaphoreType.DMA,
        ],
    )
    def agg_kernel(h_hbm, src_hbm, dst_hbm, z_hbm, out_hbm,
                   acc, sidx, didx, buf0, buf1, sem0, sem1):
        cid = lax.axis_index("c")
        sid = lax.axis_index("s")
        wid = sid * NC + cid
        r0 = sid * ROWS_PT
        pltpu.sync_copy(z_hbm.at[pl.ds(r0, ROWS_PT)], acc.at[pl.ds(r0, ROWS_PT)])
        plsc.subcore_barrier()

        def _wait(buf, sem):
            pltpu.make_async_copy(h_hbm.at[pl.ds(0, K)], buf, sem).wait()

        def _issue(sidx_ref, c, buf, sem):
            for q in range(NSPLIT):
                pltpu.async_copy(
                    h_hbm.at[sidx_ref.at[c, pl.ds(q * QS, QS)]],
                    buf.at[pl.ds(q * QS, QS)], sem)

        for h in (0, 1):
            pltpu.sync_copy(src_hbm.at[wid, pl.ds(h * CH, CH)], sidx)
            pltpu.sync_copy(dst_hbm.at[wid, pl.ds(h * CH, CH)], didx)
            _issue(sidx, 0, buf0, sem0)

            @pl.loop(0, CH, step=2)
            def _(c):
                _wait(buf0, sem0)
                _issue(sidx, c + 1, buf1, sem1)
                pltpu.sync_copy(buf0, acc.at[didx.at[c]], add=True)
                _wait(buf1, sem1)

                @pl.when(c + 2 < CH)
                def _():
                    _issue(sidx, c + 2, buf0, sem0)

                pltpu.sync_copy(buf1, acc.at[didx.at[c + 1]], add=True)

        plsc.subcore_barrier()
        pltpu.sync_copy(acc.at[pl.ds(r0, ROWS_PT)],
                        out_hbm.at[cid, pl.ds(r0, ROWS_PT)])

    return agg_kernel(h_pad, src_w, dst_w, zeros128)


def _dis_block(degp_ref):
    deg = jnp.sum(degp_ref[...], axis=0)[:, None] + 1.0
    return lax.rsqrt(deg)


def _mm_body(x_ref, w_ref, o_ref):
    o_ref[...] = jnp.dot(x_ref[...], w_ref[...],
                         preferred_element_type=jnp.float32)


def _scale_body(h_ref, degp_ref, o_ref):
    o_ref[...] = h_ref[...] * _dis_block(degp_ref)


def _mid_body(aggp_ref, hp_ref, degp_ref, b_ref, w_ref, o_ref):
    dis = _dis_block(degp_ref)
    z = (aggp_ref[0] + aggp_ref[1] + hp_ref[...]) * dis + b_ref[...]
    z = jnp.maximum(z, 0.0)
    o_ref[...] = jnp.dot(z, w_ref[...],
                         preferred_element_type=jnp.float32) * dis


def _out_body(aggp_ref, hp_ref, degp_ref, b_ref, w_ref, bo_ref, o_ref):
    dis = _dis_block(degp_ref)
    z = (aggp_ref[0] + aggp_ref[1] + hp_ref[...]) * dis + b_ref[...]
    logits = jnp.dot(z, w_ref[...],
                     preferred_element_type=jnp.float32) + bo_ref[...]
    m = jnp.max(logits, axis=1, keepdims=True)
    e = jnp.exp(logits - m)
    o_ref[...] = e / jnp.sum(e, axis=1, keepdims=True)


_GRID = (N_PAD // R_BLK,)
_ROW = pl.BlockSpec((R_BLK, D), lambda i: (i, 0))
_DEGP = pl.BlockSpec((NW, R_BLK), lambda i: (0, i))
_AGGP = pl.BlockSpec((NC, R_BLK, D), lambda i: (0, i, 0))
_WFULL = pl.BlockSpec((D, HIDDEN), lambda i: (0, 0))
_BROW = pl.BlockSpec((1, HIDDEN), lambda i: (0, 0))


def _tc_matmul(x_pad, W):
    return pl.pallas_call(
        _mm_body, grid=_GRID,
        in_specs=[_ROW, _WFULL], out_specs=_ROW,
        out_shape=jax.ShapeDtypeStruct((N_PAD, HIDDEN), jnp.float32),
    )(x_pad, W)


def _tc_scale(h, degp):
    return pl.pallas_call(
        _scale_body, grid=_GRID,
        in_specs=[_ROW, _DEGP], out_specs=_ROW,
        out_shape=jax.ShapeDtypeStruct((N_PAD, HIDDEN), jnp.float32),
    )(h, degp)


def _tc_mid(aggp, hp, degp, b, W):
    return pl.pallas_call(
        _mid_body, grid=_GRID,
        in_specs=[_AGGP, _ROW, _DEGP, _BROW, _WFULL], out_specs=_ROW,
        out_shape=jax.ShapeDtypeStruct((N_PAD, HIDDEN), jnp.float32),
    )(aggp, hp, degp, b, W)


def _tc_out(aggp, hp, degp, b, W_out, b_out):
    return pl.pallas_call(
        _out_body, grid=_GRID,
        in_specs=[_AGGP, _ROW, _DEGP, _BROW,
                  pl.BlockSpec((HIDDEN, N_CLASSES), lambda i: (0, 0)),
                  pl.BlockSpec((1, N_CLASSES), lambda i: (0, 0))],
        out_specs=pl.BlockSpec((R_BLK, N_CLASSES), lambda i: (i, 0)),
        out_shape=jax.ShapeDtypeStruct((N_PAD, N_CLASSES), jnp.float32),
    )(aggp, hp, degp, b, W_out, b_out)


def kernel(x, edge_index, W1, b1, W2, b2, W_out, b_out):
    src = edge_index[0].astype(jnp.int32)
    dst = edge_index[1].astype(jnp.int32)
    E = src.shape[0]
    C = 2 * CH
    assert NW * C * K >= E
    pad = NW * C * K - E
    n_dummy = N_PAD - N_NODES
    assert pad % n_dummy == 0
    fill = (N_NODES + jnp.broadcast_to(
        jnp.arange(n_dummy, dtype=jnp.int32), (pad // n_dummy, n_dummy))
            ).reshape(pad // K, K)
    src_w = jnp.concatenate([src.reshape(E // K, K), fill]).reshape(NW, C, K)
    dst_w = jnp.concatenate([dst.reshape(E // K, K), fill]).reshape(NW, C, K)
    x_pad = jnp.concatenate(
        [x, jnp.zeros((N_PAD - N_NODES, D), jnp.float32)])
    zeros128 = jnp.zeros((N_PAD, D), jnp.float32)
    b1r = b1.reshape(1, HIDDEN)
    b2r = b2.reshape(1, HIDDEN)
    boutr = b_out.reshape(1, N_CLASSES)

    degp = _sc_degree(dst_w)
    h1 = _tc_matmul(x_pad, W1)
    h1p = _tc_scale(h1, degp)
    agg1 = _sc_aggregate(h1p, src_w, dst_w, zeros128)
    h2p = _tc_mid(agg1, h1p, degp, b1r, W2)
    agg2 = _sc_aggregate(h2p, src_w, dst_w, zeros128)
    outp = _tc_out(agg2, h2p, degp, b2r, W_out, boutr)
    return outp[:N_NODES]


# speedup vs baseline: 1.0208x; 1.0208x over previous
import dataclasses
import functools

import jax
import jax.numpy as jnp
from jax import lax
from jax.experimental import pallas as pl
from jax.experimental.pallas import tpu as pltpu
from jax.experimental.pallas import tpu_sc as plsc

N_NODES = 10000
D = 128
HIDDEN = 128
N_CLASSES = 40

NC = 2
NS = 16
NW = NC * NS
K = 128
CH = 40
NSPLIT = 4
QS = K // NSPLIT
N_PAD = 10240
ROWS_PT = N_PAD // NS
R_BLK = 2560

_MESH = plsc.VectorSubcoreMesh(core_axis_name="c", subcore_axis_name="s")

_CP_NO_LAYOUT = pltpu.CompilerParams()
if "needs_layout_passes" in pltpu.CompilerParams.__dataclass_fields__:
    _CP_NO_LAYOUT = dataclasses.replace(_CP_NO_LAYOUT, needs_layout_passes=False)


def _sc_degree(dst_w):
    C = dst_w.shape[1]

    @functools.partial(
        pl.kernel,
        out_type=jax.ShapeDtypeStruct((NW, N_PAD), jnp.float32),
        mesh=_MESH,
        compiler_params=_CP_NO_LAYOUT,
        scratch_types=[
            pltpu.VMEM((N_PAD,), jnp.float32),
            pltpu.VMEM((C, K), jnp.int32),
        ],
    )
    def deg_kernel(dst_hbm, out_hbm, deg_v, idx_v):
        cid = lax.axis_index("c")
        sid = lax.axis_index("s")
        wid = sid * NC + cid
        pltpu.sync_copy(dst_hbm.at[wid], idx_v)

        @pl.loop(0, N_PAD // 16)
        def _(i):
            deg_v[pl.ds(i * 16, 16)] = jnp.zeros((16,), jnp.float32)

        ones = jnp.ones((16,), jnp.float32)

        @pl.loop(0, C)
        def _(c):
            for j in range(K // 16):
                idx = idx_v[c, pl.ds(j * 16, 16)]
                plsc.addupdate_scatter(deg_v, [idx], ones)

        pltpu.sync_copy(deg_v, out_hbm.at[wid])

    return deg_kernel(dst_w)


def _sc_aggregate(h_pad, src_w, dst_w):
    C = src_w.shape[1]
    assert C == 2 * CH

    @functools.partial(
        pl.kernel,
        out_type=jax.ShapeDtypeStruct((NC, N_PAD, D), jnp.float32),
        mesh=_MESH,
        scratch_types=[
            pltpu.VMEM_SHARED((N_PAD, D), jnp.float32),
            pltpu.VMEM((CH, K), jnp.int32),
            pltpu.VMEM((CH, K), jnp.int32),
            pltpu.VMEM((K, D), jnp.float32),
            pltpu.VMEM((K, D), jnp.float32),
            pltpu.SemaphoreType.DMA,
            pltpu.SemaphoreType.DMA,
        ],
    )
    def agg_kernel(h_hbm, src_hbm, dst_hbm, out_hbm,
                   acc, sidx, didx, buf0, buf1, sem0, sem1):
        cid = lax.axis_index("c")
        sid = lax.axis_index("s")
        wid = sid * NC + cid
        r0 = sid * ROWS_PT

        @pl.loop(0, K)
        def _(r):
            for j in range(D // 16):
                buf0[r, pl.ds(j * 16, 16)] = jnp.zeros((16,), jnp.float32)

        for b in range(ROWS_PT // K):
            pltpu.sync_copy(buf0, acc.at[pl.ds(r0 + b * K, K)])
        plsc.subcore_barrier()

        def _wait(buf, sem):
            pltpu.make_async_copy(h_hbm.at[pl.ds(0, K)], buf, sem).wait()

        def _issue(sidx_ref, c, buf, sem):
            for q in range(NSPLIT):
                pltpu.async_copy(
                    h_hbm.at[sidx_ref.at[c, pl.ds(q * QS, QS)]],
                    buf.at[pl.ds(q * QS, QS)], sem)

        for h in (0, 1):
            pltpu.sync_copy(src_hbm.at[wid, pl.ds(h * CH, CH)], sidx)
            pltpu.sync_copy(dst_hbm.at[wid, pl.ds(h * CH, CH)], didx)
            _issue(sidx, 0, buf0, sem0)

            @pl.loop(0, CH, step=2)
            def _(c):
                _wait(buf0, sem0)
                _issue(sidx, c + 1, buf1, sem1)
                pltpu.sync_copy(buf0, acc.at[didx.at[c]], add=True)
                _wait(buf1, sem1)

                @pl.when(c + 2 < CH)
                def _():
                    _issue(sidx, c + 2, buf0, sem0)

                pltpu.sync_copy(buf1, acc.at[didx.at[c + 1]], add=True)

        plsc.subcore_barrier()
        pltpu.sync_copy(acc.at[pl.ds(r0, ROWS_PT)],
                        out_hbm.at[cid, pl.ds(r0, ROWS_PT)])

    return agg_kernel(h_pad, src_w, dst_w)


def _dis_block(degp_ref):
    deg = jnp.sum(degp_ref[...], axis=0)[:, None] + 1.0
    return lax.rsqrt(deg)


def _mm_body(x_ref, w_ref, o_ref):
    o_ref[...] = jnp.dot(x_ref[...], w_ref[...],
                         preferred_element_type=jnp.float32)


def _scale_body(h_ref, degp_ref, o_ref):
    o_ref[...] = h_ref[...] * _dis_block(degp_ref)


def _mid_body(aggp_ref, hp_ref, degp_ref, b_ref, w_ref, o_ref):
    dis = _dis_block(degp_ref)
    z = (aggp_ref[0] + aggp_ref[1] + hp_ref[...]) * dis + b_ref[...]
    z = jnp.maximum(z, 0.0)
    o_ref[...] = jnp.dot(z, w_ref[...],
                         preferred_element_type=jnp.float32) * dis


def _out_body(aggp_ref, hp_ref, degp_ref, b_ref, w_ref, bo_ref, o_ref):
    dis = _dis_block(degp_ref)
    z = (aggp_ref[0] + aggp_ref[1] + hp_ref[...]) * dis + b_ref[...]
    logits = jnp.dot(z, w_ref[...],
                     preferred_element_type=jnp.float32) + bo_ref[...]
    m = jnp.max(logits, axis=1, keepdims=True)
    e = jnp.exp(logits - m)
    o_ref[...] = e / jnp.sum(e, axis=1, keepdims=True)


_GRID = (N_PAD // R_BLK,)
_ROW = pl.BlockSpec((R_BLK, D), lambda i: (i, 0))
_DEGP = pl.BlockSpec((NW, R_BLK), lambda i: (0, i))
_AGGP = pl.BlockSpec((NC, R_BLK, D), lambda i: (0, i, 0))
_WFULL = pl.BlockSpec((D, HIDDEN), lambda i: (0, 0))
_BROW = pl.BlockSpec((1, HIDDEN), lambda i: (0, 0))


def _tc_matmul(x_pad, W):
    return pl.pallas_call(
        _mm_body, grid=_GRID,
        in_specs=[_ROW, _WFULL], out_specs=_ROW,
        out_shape=jax.ShapeDtypeStruct((N_PAD, HIDDEN), jnp.float32),
    )(x_pad, W)


def _tc_scale(h, degp):
    return pl.pallas_call(
        _scale_body, grid=_GRID,
        in_specs=[_ROW, _DEGP], out_specs=_ROW,
        out_shape=jax.ShapeDtypeStruct((N_PAD, HIDDEN), jnp.float32),
    )(h, degp)


def _tc_mid(aggp, hp, degp, b, W):
    return pl.pallas_call(
        _mid_body, grid=_GRID,
        in_specs=[_AGGP, _ROW, _DEGP, _BROW, _WFULL], out_specs=_ROW,
        out_shape=jax.ShapeDtypeStruct((N_PAD, HIDDEN), jnp.float32),
    )(aggp, hp, degp, b, W)


def _tc_out(aggp, hp, degp, b, W_out, b_out):
    return pl.pallas_call(
        _out_body, grid=_GRID,
        in_specs=[_AGGP, _ROW, _DEGP, _BROW,
                  pl.BlockSpec((HIDDEN, N_CLASSES), lambda i: (0, 0)),
                  pl.BlockSpec((1, N_CLASSES), lambda i: (0, 0))],
        out_specs=pl.BlockSpec((R_BLK, N_CLASSES), lambda i: (i, 0)),
        out_shape=jax.ShapeDtypeStruct((N_PAD, N_CLASSES), jnp.float32),
    )(aggp, hp, degp, b, W_out, b_out)


def kernel(x, edge_index, W1, b1, W2, b2, W_out, b_out):
    src = edge_index[0].astype(jnp.int32)
    dst = edge_index[1].astype(jnp.int32)
    E = src.shape[0]
    C = 2 * CH
    assert NW * C * K >= E
    pad = NW * C * K - E
    n_dummy = N_PAD - N_NODES
    assert pad % n_dummy == 0
    fill = (N_NODES + jnp.broadcast_to(
        jnp.arange(n_dummy, dtype=jnp.int32), (pad // n_dummy, n_dummy))
            ).reshape(pad // K, K)
    src_w = jnp.concatenate([src.reshape(E // K, K), fill]).reshape(NW, C, K)
    dst_w = jnp.concatenate([dst.reshape(E // K, K), fill]).reshape(NW, C, K)
    x_pad = jnp.concatenate(
        [x, jnp.zeros((N_PAD - N_NODES, D), jnp.float32)])
    b1r = b1.reshape(1, HIDDEN)
    b2r = b2.reshape(1, HIDDEN)
    boutr = b_out.reshape(1, N_CLASSES)

    degp = _sc_degree(dst_w)
    h1 = _tc_matmul(x_pad, W1)
    h1p = _tc_scale(h1, degp)
    agg1 = _sc_aggregate(h1p, src_w, dst_w)
    h2p = _tc_mid(agg1, h1p, degp, b1r, W2)
    agg2 = _sc_aggregate(h2p, src_w, dst_w)
    outp = _tc_out(agg2, h2p, degp, b2r, W_out, boutr)
    return outp[:N_NODES]
